# baseline (speedup 1.0000x reference)
import functools

import jax
import jax.numpy as jnp
from jax import lax
from jax.experimental import pallas as pl
from jax.experimental.pallas import tpu as pltpu
from jax.experimental.pallas import tpu_sc as plsc

_NC = 2
_NS = 16
_NW = _NC * _NS
_L = 16

_V = 1000
_D = 1000
_B = 1024
_S = 50

_NVG = _D // 8
_NBG = _B // 128
_NITEMS = _NVG * _S
_IPW = -(-_NITEMS // _NW)


def _gather_body(idx_hbm, table_hbm, y5_hbm, idx_v, slab_v, obuf0, obuf1,
                 ssem0, ssem1):
    wid = lax.axis_index("s") * _NC + lax.axis_index("c")
    start = wid * _IPW

    pltpu.sync_copy(idx_hbm, idx_v)

    obufs = (obuf0, obuf1)
    ssems = (ssem0, ssem1)

    def do_item(j, prev_vg, obuf, ssem):
        item = lax.min(start + j, _NITEMS - 1)
        vg = item // _S
        s = item - vg * _S

        @pl.when(vg != prev_vg)
        def _load_slab():
            pltpu.sync_copy(table_hbm.at[:, pl.ds(vg * 8, 8)], slab_v)

        for bg in range(_NBG):
            toks = [idx_v[s, bg, pl.ds(c * _L, _L)] for c in range(8)]
            for vr in range(8):
                vrv = jnp.full((_L,), vr, dtype=jnp.int32)
                for c in range(8):
                    val = plsc.load_gather(slab_v, [toks[c], vrv])
                    obuf[bg, vr, pl.ds(c * _L, _L)] = val

        pltpu.async_copy(obuf, y5_hbm.at[s, vg], ssem)
        return vg

    def body(p, prev_vg):
        for b in range(2):
            j = 2 * p + b

            @pl.when(p >= 1)
            def _wait_prev():
                pltpu.make_async_copy(
                    obufs[b], y5_hbm.at[0, 0], ssems[b]
                ).wait()

            prev_vg = do_item(j, prev_vg, obufs[b], ssems[b])
        return prev_vg

    lax.fori_loop(0, _IPW // 2, body, jnp.int32(-1))

    pltpu.make_async_copy(obuf0, y5_hbm.at[0, 0], ssem0).wait()
    pltpu.make_async_copy(obuf1, y5_hbm.at[0, 0], ssem1).wait()


_mesh = plsc.VectorSubcoreMesh(
    core_axis_name="c", subcore_axis_name="s",
    num_cores=_NC, num_subcores=_NS,
)

_gather_call = functools.partial(
    pl.kernel,
    out_type=jax.ShapeDtypeStruct((_S, _NVG, _NBG, 8, 128), jnp.float32),
    mesh=_mesh,
    compiler_params=pltpu.CompilerParams(
        use_tc_tiling_on_sc=False, needs_layout_passes=False
    ),
    scratch_types=[
        pltpu.VMEM((_S, _NBG, 128), jnp.int32),
        pltpu.VMEM((_V, 8), jnp.float32),
        pltpu.VMEM((_NBG, 8, 128), jnp.float32),
        pltpu.VMEM((_NBG, 8, 128), jnp.float32),
        pltpu.SemaphoreType.DMA,
        pltpu.SemaphoreType.DMA,
    ],
)(_gather_body)


@jax.jit
def kernel(idx, table):
    idx_t = idx.T.reshape(_S, _NBG, 128).astype(jnp.int32)
    y5 = _gather_call(idx_t, table)
    return y5.transpose(2, 4, 0, 1, 3).reshape(_B, _S, _D)


# speedup vs baseline: 2.1740x; 2.1740x over previous
import functools

import jax
import jax.numpy as jnp
from jax import lax
from jax.experimental import pallas as pl
from jax.experimental.pallas import tpu as pltpu
from jax.experimental.pallas import tpu_sc as plsc

_NC = 2
_NS = 16
_NW = _NC * _NS
_L = 16

_V = 1000
_D = 1000
_B = 1024
_S = 50

_NVG = _D // 8
_NBG = _B // 128
_NITEMS = _NVG * _S
_IPW = -(-_NITEMS // _NW)


def _gather_body(idx_hbm, table_hbm, y5_hbm, idx_v, slab_v, obuf0, obuf1,
                 ssem0, ssem1):
    wid = lax.axis_index("s") * _NC + lax.axis_index("c")
    start = wid * _IPW

    pltpu.sync_copy(idx_hbm, idx_v)

    obufs = (obuf0, obuf1)
    ssems = (ssem0, ssem1)

    def do_item(j, prev_vg, obuf, ssem):
        item = lax.min(start + j, _NITEMS - 1)
        vg = item // _S
        s = item - vg * _S

        @pl.when(vg != prev_vg)
        def _load_slab():
            pltpu.sync_copy(table_hbm.at[:, pl.ds(vg * 8, 8)], slab_v)

        for bg in range(_NBG):
            toks = [idx_v[s, bg, pl.ds(c * _L, _L)] for c in range(8)]

            @plsc.parallel_loop(0, 8, unroll=2)
            def _rows(vr):
                vrv = jnp.full((_L,), vr, dtype=jnp.int32)
                for c in range(8):
                    val = plsc.load_gather(slab_v, [toks[c], vrv])
                    obuf[bg, vr, pl.ds(c * _L, _L)] = val

        pltpu.async_copy(obuf, y5_hbm.at[s, vg], ssem)
        return vg

    def body(p, prev_vg):
        for b in range(2):
            j = 2 * p + b

            @pl.when(p >= 1)
            def _wait_prev():
                pltpu.make_async_copy(
                    obufs[b], y5_hbm.at[0, 0], ssems[b]
                ).wait()

            prev_vg = do_item(j, prev_vg, obufs[b], ssems[b])
        return prev_vg

    lax.fori_loop(0, _IPW // 2, body, jnp.int32(-1))

    pltpu.make_async_copy(obuf0, y5_hbm.at[0, 0], ssem0).wait()
    pltpu.make_async_copy(obuf1, y5_hbm.at[0, 0], ssem1).wait()


_mesh = plsc.VectorSubcoreMesh(
    core_axis_name="c", subcore_axis_name="s",
    num_cores=_NC, num_subcores=_NS,
)

_gather_call = functools.partial(
    pl.kernel,
    out_type=jax.ShapeDtypeStruct((_S, _NVG, _NBG, 8, 128), jnp.float32),
    mesh=_mesh,
    compiler_params=pltpu.CompilerParams(
        use_tc_tiling_on_sc=False, needs_layout_passes=False
    ),
    scratch_types=[
        pltpu.VMEM((_S, _NBG, 128), jnp.int32),
        pltpu.VMEM((_V, 8), jnp.float32),
        pltpu.VMEM((_NBG, 8, 128), jnp.float32),
        pltpu.VMEM((_NBG, 8, 128), jnp.float32),
        pltpu.SemaphoreType.DMA,
        pltpu.SemaphoreType.DMA,
    ],
)(_gather_body)


@jax.jit
def kernel(idx, table):
    idx_t = idx.T.reshape(_S, _NBG, 128).astype(jnp.int32)
    y5 = _gather_call(idx_t, table)
    return y5.transpose(2, 4, 0, 1, 3).reshape(_B, _S, _D)


# speedup vs baseline: 3.3065x; 1.5209x over previous
import functools

import jax
import jax.numpy as jnp
from jax import lax
from jax.experimental import pallas as pl
from jax.experimental.pallas import tpu as pltpu
from jax.experimental.pallas import tpu_sc as plsc

_NC = 2
_NS = 16
_NW = _NC * _NS
_L = 16

_V = 1000
_D = 1000
_B = 1024
_S = 50

_NVG = _D // 8
_NBG = _B // 128
_NITEMS = _NVG * _S
_IPW = -(-_NITEMS // _NW)


def _gather_body(idx_hbm, table_hbm, y5_hbm, idx_v, slab_v, obuf0, obuf1,
                 ssem0, ssem1):
    wid = lax.axis_index("s") * _NC + lax.axis_index("c")
    start = wid * _IPW

    pltpu.sync_copy(idx_hbm, idx_v)

    obufs = (obuf0, obuf1)
    ssems = (ssem0, ssem1)

    def do_item(j, prev_vg, obuf, ssem):
        item = lax.min(start + j, _NITEMS - 1)
        vg = item // _S
        s = item - vg * _S

        @pl.when(vg != prev_vg)
        def _load_slab():
            pltpu.sync_copy(table_hbm.at[:, pl.ds(vg * 8, 8)], slab_v)

        for bg in range(_NBG):
            toks = [idx_v[s, bg, pl.ds(c * _L, _L)] for c in range(8)]

            @plsc.parallel_loop(0, 8, unroll=4)
            def _rows(vr):
                vrv = jnp.full((_L,), vr, dtype=jnp.int32)
                for c in range(8):
                    val = plsc.load_gather(slab_v, [toks[c], vrv])
                    obuf[bg, vr, pl.ds(c * _L, _L)] = val

        pltpu.async_copy(obuf, y5_hbm.at[s, vg], ssem)
        return vg

    def body(p, prev_vg):
        for b in range(2):
            j = 2 * p + b

            @pl.when(p >= 1)
            def _wait_prev():
                pltpu.make_async_copy(
                    obufs[b], y5_hbm.at[0, 0], ssems[b]
                ).wait()

            prev_vg = do_item(j, prev_vg, obufs[b], ssems[b])
        return prev_vg

    lax.fori_loop(0, _IPW // 2, body, jnp.int32(-1))

    pltpu.make_async_copy(obuf0, y5_hbm.at[0, 0], ssem0).wait()
    pltpu.make_async_copy(obuf1, y5_hbm.at[0, 0], ssem1).wait()


_mesh = plsc.VectorSubcoreMesh(
    core_axis_name="c", subcore_axis_name="s",
    num_cores=_NC, num_subcores=_NS,
)

_gather_call = functools.partial(
    pl.kernel,
    out_type=jax.ShapeDtypeStruct((_S, _NVG, _NBG, 8, 128), jnp.float32),
    mesh=_mesh,
    compiler_params=pltpu.CompilerParams(
        use_tc_tiling_on_sc=False, needs_layout_passes=False
    ),
    scratch_types=[
        pltpu.VMEM((_S, _NBG, 128), jnp.int32),
        pltpu.VMEM((_V, 8), jnp.float32),
        pltpu.VMEM((_NBG, 8, 128), jnp.float32),
        pltpu.VMEM((_NBG, 8, 128), jnp.float32),
        pltpu.SemaphoreType.DMA,
        pltpu.SemaphoreType.DMA,
    ],
)(_gather_body)


@jax.jit
def kernel(idx, table):
    idx_t = idx.T.reshape(_S, _NBG, 128).astype(jnp.int32)
    y5 = _gather_call(idx_t, table)
    return y5.transpose(2, 4, 0, 1, 3).reshape(_B, _S, _D)
